# Initial kernel scaffold; baseline (speedup 1.0000x reference)
import jax
import jax.numpy as jnp
from jax.experimental import pallas as pl


def kernel(pred, target):
    raise NotImplementedError("write your pallas kernel here")


# speedup vs baseline: 25.4157x; 25.4157x over previous
import functools

import jax
import jax.numpy as jnp
from jax.experimental import pallas as pl
from jax.experimental.pallas import tpu as pltpu

_NUM_CLASSES = 21
_SMOOTH = 1e-06


def _acc_body(pred_ref, tgt_ref, inter_ref, union_ref, *, rows):
    s = pl.program_id(1)

    @pl.when(s == 0)
    def _():
        inter_ref[...] = jnp.zeros_like(inter_ref)
        union_ref[...] = jnp.zeros_like(union_ref)

    for r in range(rows // 8):
        p8 = pred_ref[0, :, r * 8:(r + 1) * 8, :]
        t8 = tgt_ref[0, r * 8:(r + 1) * 8, :]
        maxv = jnp.max(p8, axis=0)
        for c in range(_NUM_CLASSES):
            pc = p8[c] == maxv
            tc = t8 == c
            inter_ref[0, c] += jnp.where(pc & tc, 1.0, 0.0)
            union_ref[0, c] += jnp.where(pc | tc, 1.0, 0.0)


def _fin_body(inter_ref, union_ref, out_ref):
    isum = jnp.sum(inter_ref[...], axis=(0, 2, 3))
    usum = jnp.sum(union_ref[...], axis=(0, 2, 3))
    iou = (isum + _SMOOTH) / (usum + _SMOOTH)
    out_ref[...] = jnp.broadcast_to(jnp.mean(iou), out_ref.shape)


def kernel(pred, target):
    B, C, H, W = pred.shape
    rows = 128
    cpb = H // rows
    cores = 2
    bpc = B // cores
    inner = bpc * cpb

    out_sds = [jax.ShapeDtypeStruct((cores, C, 8, W), jnp.float32)] * 2
    inter, union = pl.pallas_call(
        functools.partial(_acc_body, rows=rows),
        grid=(cores, inner),
        in_specs=[
            pl.BlockSpec((1, C, rows, W),
                         lambda c, s: (c * bpc + s // cpb, 0, s % cpb, 0)),
            pl.BlockSpec((1, rows, W),
                         lambda c, s: (c * bpc + s // cpb, s % cpb, 0)),
        ],
        out_specs=[pl.BlockSpec((1, C, 8, W), lambda c, s: (c, 0, 0, 0))] * 2,
        out_shape=out_sds,
        compiler_params=pltpu.CompilerParams(
            dimension_semantics=("parallel", "arbitrary")),
        name="iou_counts",
    )(pred, target)

    out = pl.pallas_call(
        _fin_body,
        out_shape=jax.ShapeDtypeStruct((8, 128), jnp.float32),
        name="iou_finalize",
    )(inter, union)
    return out[0, 0]


# speedup vs baseline: 28.1378x; 1.1071x over previous
import functools

import jax
import jax.numpy as jnp
from jax.experimental import pallas as pl
from jax.experimental.pallas import tpu as pltpu

_NUM_CLASSES = 21
_SMOOTH = 1e-06


def _acc_body(pred_ref, tgt_ref, inter_ref, union_ref, *, rows):
    s = pl.program_id(1)

    @pl.when(s == 0)
    def _():
        inter_ref[...] = jnp.zeros_like(inter_ref)
        union_ref[...] = jnp.zeros_like(union_ref)

    for r in range(rows // 8):
        p8 = pred_ref[0, :, r * 8:(r + 1) * 8, :]
        t8 = tgt_ref[0, r * 8:(r + 1) * 8, :]
        maxv = jnp.max(p8, axis=0)
        for c in range(_NUM_CLASSES):
            pc = p8[c] == maxv
            tc = t8 == c
            inter_ref[0, c] += jnp.where(pc & tc, 1.0, 0.0)
            union_ref[0, c] += jnp.where(pc | tc, 1.0, 0.0)


def _fin_body(inter_ref, union_ref, out_ref):
    isum = jnp.sum(inter_ref[...], axis=(0, 2, 3))
    usum = jnp.sum(union_ref[...], axis=(0, 2, 3))
    iou = (isum + _SMOOTH) / (usum + _SMOOTH)
    out_ref[...] = jnp.broadcast_to(jnp.mean(iou), out_ref.shape)


def kernel(pred, target):
    B, C, H, W = pred.shape
    rows = 256
    cpb = H // rows
    cores = 2
    bpc = B // cores
    inner = bpc * cpb

    out_sds = [jax.ShapeDtypeStruct((cores, C, 8, W), jnp.float32)] * 2
    inter, union = pl.pallas_call(
        functools.partial(_acc_body, rows=rows),
        grid=(cores, inner),
        in_specs=[
            pl.BlockSpec((1, C, rows, W),
                         lambda c, s: (c * bpc + s // cpb, 0, s % cpb, 0)),
            pl.BlockSpec((1, rows, W),
                         lambda c, s: (c * bpc + s // cpb, s % cpb, 0)),
        ],
        out_specs=[pl.BlockSpec((1, C, 8, W), lambda c, s: (c, 0, 0, 0))] * 2,
        out_shape=out_sds,
        compiler_params=pltpu.CompilerParams(
            dimension_semantics=("parallel", "arbitrary")),
        name="iou_counts",
    )(pred, target)

    out = pl.pallas_call(
        _fin_body,
        out_shape=jax.ShapeDtypeStruct((8, 128), jnp.float32),
        name="iou_finalize",
    )(inter, union)
    return out[0, 0]


# speedup vs baseline: 28.7508x; 1.0218x over previous
import functools

import jax
import jax.numpy as jnp
from jax.experimental import pallas as pl
from jax.experimental.pallas import tpu as pltpu

_NUM_CLASSES = 21
_SMOOTH = 1e-06


def _acc_body(pred_ref, tgt_ref, inter_ref, union_ref, *, rows):
    s = pl.program_id(1)

    @pl.when(s == 0)
    def _():
        inter_ref[...] = jnp.zeros_like(inter_ref)
        union_ref[...] = jnp.zeros_like(union_ref)

    for r in range(rows // 8):
        p8 = pred_ref[0, :, r * 8:(r + 1) * 8, :]
        t8 = tgt_ref[0, r * 8:(r + 1) * 8, :]
        maxv = jnp.max(p8, axis=0)
        for c in range(_NUM_CLASSES):
            pc = p8[c] == maxv
            tc = t8 == c
            inter_ref[0, c] += jnp.where(pc & tc, 1.0, 0.0)
            union_ref[0, c] += jnp.where(pc | tc, 1.0, 0.0)


def _fin_body(inter_ref, union_ref, out_ref):
    isum = jnp.sum(inter_ref[...], axis=(0, 2, 3))
    usum = jnp.sum(union_ref[...], axis=(0, 2, 3))
    iou = (isum + _SMOOTH) / (usum + _SMOOTH)
    out_ref[...] = jnp.broadcast_to(jnp.mean(iou), out_ref.shape)


def kernel(pred, target):
    B, C, H, W = pred.shape
    rows = 512
    cpb = H // rows
    cores = 2
    bpc = B // cores
    inner = bpc * cpb

    out_sds = [jax.ShapeDtypeStruct((cores, C, 8, W), jnp.float32)] * 2
    inter, union = pl.pallas_call(
        functools.partial(_acc_body, rows=rows),
        grid=(cores, inner),
        in_specs=[
            pl.BlockSpec((1, C, rows, W),
                         lambda c, s: (c * bpc + s // cpb, 0, s % cpb, 0)),
            pl.BlockSpec((1, rows, W),
                         lambda c, s: (c * bpc + s // cpb, s % cpb, 0)),
        ],
        out_specs=[pl.BlockSpec((1, C, 8, W), lambda c, s: (c, 0, 0, 0))] * 2,
        out_shape=out_sds,
        compiler_params=pltpu.CompilerParams(
            dimension_semantics=("parallel", "arbitrary")),
        name="iou_counts",
    )(pred, target)

    out = pl.pallas_call(
        _fin_body,
        out_shape=jax.ShapeDtypeStruct((8, 128), jnp.float32),
        name="iou_finalize",
    )(inter, union)
    return out[0, 0]


# speedup vs baseline: 29.4314x; 1.0237x over previous
import functools

import jax
import jax.numpy as jnp
from jax.experimental import pallas as pl
from jax.experimental.pallas import tpu as pltpu

_NUM_CLASSES = 21
_SMOOTH = 1e-06


def _acc_body(pred_ref, tgt_ref, inter_ref, union_ref, *, rows):
    s = pl.program_id(0)

    @pl.when(s == 0)
    def _():
        inter_ref[...] = jnp.zeros_like(inter_ref)
        union_ref[...] = jnp.zeros_like(union_ref)

    for r in range(rows // 16):
        sl = slice(r * 16, r * 16 + 16)
        t16 = tgt_ref[0, sl, :]
        maxv = pred_ref[0, 0, sl, :]
        for c in range(1, _NUM_CLASSES):
            maxv = jnp.maximum(maxv, pred_ref[0, c, sl, :])
        for c in range(_NUM_CLASSES):
            pc = pred_ref[0, c, sl, :] == maxv
            tc_f = jnp.where(t16 == c, 1.0, 0.0)
            i_f = jnp.where(pc, tc_f, 0.0)
            u_f = jnp.where(pc, 1.0, tc_f)
            inter_ref[c] += i_f[0:8] + i_f[8:16]
            union_ref[c] += u_f[0:8] + u_f[8:16]


def _fin_body(inter_ref, union_ref, out_ref):
    isum = jnp.sum(inter_ref[...], axis=(1, 2))
    usum = jnp.sum(union_ref[...], axis=(1, 2))
    iou = (isum + _SMOOTH) / (usum + _SMOOTH)
    out_ref[...] = jnp.broadcast_to(jnp.mean(iou), out_ref.shape)


def kernel(pred, target):
    B, C, H, W = pred.shape
    rows = 512
    cpb = H // rows
    steps = B * cpb

    out_sds = [jax.ShapeDtypeStruct((C, 8, W), jnp.float32)] * 2
    inter, union = pl.pallas_call(
        functools.partial(_acc_body, rows=rows),
        grid=(steps,),
        in_specs=[
            pl.BlockSpec((1, C, rows, W), lambda s: (s // cpb, 0, s % cpb, 0)),
            pl.BlockSpec((1, rows, W), lambda s: (s // cpb, s % cpb, 0)),
        ],
        out_specs=[pl.BlockSpec((C, 8, W), lambda s: (0, 0, 0))] * 2,
        out_shape=out_sds,
        compiler_params=pltpu.CompilerParams(
            dimension_semantics=("arbitrary",)),
        name="iou_counts",
    )(pred, target)

    out = pl.pallas_call(
        _fin_body,
        out_shape=jax.ShapeDtypeStruct((8, 128), jnp.float32),
        name="iou_finalize",
    )(inter, union)
    return out[0, 0]
